# baseline (speedup 1.0000x reference)
import jax
import jax.numpy as jnp
from jax import lax
from jax.experimental import pallas as pl
from jax.experimental.pallas import tpu as pltpu, tpu_sc as plsc

N = 320000
S = 8
H = 128
NUM_NODES = 10000
EMB_ROWS = 2 ** (S + 1)
KEYSPACE = NUM_NODES * 256
HALF = KEYSPACE // 2
TBL = HALF + 128
KROWS = N // 128
KROWS_PAD = 2560

_GB = 4000
_GRID = N // _GB


def _counts_body(key_hbm, out_hbm, tbl, kb, idx2, ones, zb, sem):
    c = lax.axis_index("c")
    s = lax.axis_index("s")
    half = jnp.int32(HALF)
    dump = jnp.full((16,), HALF, jnp.int32)

    def _zb(i, _):
        zb[pl.ds(i * 16, 16)] = jnp.zeros((16,), jnp.int32)
        return 0
    lax.fori_loop(0, 512, _zb, 0)
    zoff = s * 80008
    zchunks = ((0, 8192), (8192, 8192), (16384, 8192), (24576, 8192),
               (32768, 8192), (40960, 8192), (49152, 8192),
               (57344, 8192), (65536, 8192), (73728, 6280))
    for off, sz in zchunks:
        pltpu.async_copy(zb.at[pl.ds(0, sz)], tbl.at[pl.ds(zoff + off, sz)],
                         sem)
    for off, sz in zchunks:
        pltpu.make_async_copy(zb.at[pl.ds(0, sz)],
                              tbl.at[pl.ds(zoff + off, sz)], sem).wait()
    for l in range(8):
        ones[pl.ds(l * 16, 16)] = jnp.ones((16,), jnp.int32)
    plsc.subcore_barrier()

    pltpu.sync_copy(key_hbm.at[pl.ds(s * 160, 160)], kb)

    def _row(j, _):
        for l in range(8):
            kv = kb[j, pl.ds(l * 16, 16)]
            lk = kv - c * half
            ok = (lk >= 0) & (lk < half)
            idx2[j, pl.ds(l * 16, 16)] = jnp.where(ok, lk, dump)
        return 0
    lax.fori_loop(0, 160, _row, 0)

    def _fire(j, _):
        pltpu.async_copy(ones, tbl.at[idx2.at[j]], sem, add=True)
        return 0
    lax.fori_loop(0, 160, _fire, 0)

    def _drain(j, _):
        pltpu.make_async_copy(ones, tbl.at[idx2.at[j]], sem).wait()
        return 0
    lax.fori_loop(0, 160, _drain, 0)
    plsc.subcore_barrier()

    pltpu.sync_copy(tbl.at[pl.ds(s * 80000, 80000)],
                    out_hbm.at[pl.ds(c * HALF + s * 80000, 80000)])


@jax.jit
def _sc_counts(key2d):
    mesh = plsc.VectorSubcoreMesh(core_axis_name="c", subcore_axis_name="s")
    return pl.kernel(
        _counts_body,
        out_type=jax.ShapeDtypeStruct((KEYSPACE,), jnp.int32),
        mesh=mesh,
        scratch_types=[
            pltpu.VMEM_SHARED((TBL,), jnp.int32),
            pltpu.VMEM((160, 128), jnp.int32),
            pltpu.VMEM((160, 128), jnp.int32),
            pltpu.VMEM((128,), jnp.int32),
            pltpu.VMEM((8192,), jnp.int32),
            pltpu.SemaphoreType.DMA,
        ],
    )(key2d)


CCAP = 10240


def _k2_body(key_hbm, ord_hbm, cnt_hbm,
             idx_hbm, ck_hbm, cd_hbm, cp_hbm, cc_hbm,
             kb, ob, cnb, cib, ib, ck0, ck1, cd0, cd1, cp0, cp1, ccv,
             cnt_smem, sem):
    c = lax.axis_index("c")
    s = lax.axis_index("s")
    w = c * 16 + s
    half = jnp.int32(HALF)
    iota = lax.iota(jnp.int32, 16)
    _B15 = jnp.full((16,), 15, jnp.int32)

    @pl.when(s == 0)
    def _():
        cnt_smem[0] = jnp.int32(0)
        cnt_smem[1] = jnp.int32(0)
    plsc.subcore_barrier()

    pltpu.sync_copy(key_hbm.at[pl.ds(w * 80, 80)], kb)
    pltpu.sync_copy(ord_hbm.at[pl.ds(w * 80, 80)], ob)

    def _cidx(j, _):
        for l in range(8):
            kv = kb[j, pl.ds(l * 16, 16)]
            cib[j, pl.ds(l * 16, 16)] = jnp.maximum(kv, 0)
        return 0
    lax.fori_loop(0, 80, _cidx, 0)

    def _fire(j, _):
        pltpu.async_copy(cnt_hbm.at[cib.at[j]], cnb.at[j], sem)
        return 0
    lax.fori_loop(0, 80, _fire, 0)

    def _drain(j, _):
        pltpu.make_async_copy(cnt_hbm.at[cib.at[j]], cnb.at[j], sem).wait()
        return 0
    lax.fori_loop(0, 80, _drain, 0)

    def _row(j, carry):
        co0, co1 = carry
        for l in range(8):
            kv = kb[j, pl.ds(l * 16, 16)]
            ov = ob[j, pl.ds(l * 16, 16)]
            cv = cnb[j, pl.ds(l * 16, 16)]
            pos = w * 10240 + j * 128 + l * 16 + iota
            valid = kv >= 0
            single = valid & (cv == 1)
            ib[j, pl.ds(l * 16, 16)] = ((kv & 255) << 1) + \
                jnp.where(single, 1, 0)
            cm = valid & (cv >= 2)
            m0 = cm & (kv < half)
            m1 = cm & (kv >= half)
            cs0 = plsc.cumsum(jnp.where(m0, 1, 0))
            offs0 = jnp.where(m0, co0 + cs0 - 1, jnp.int32(CCAP))
            plsc.store_scatter(ck0, [offs0], kv)
            plsc.store_scatter(cd0, [offs0], ov)
            plsc.store_scatter(cp0, [offs0], pos)
            co0 = co0 + jnp.take(cs0, _B15)
            cs1 = plsc.cumsum(jnp.where(m1, 1, 0))
            offs1 = jnp.where(m1, co1 + cs1 - 1, jnp.int32(CCAP))
            plsc.store_scatter(ck1, [offs1], kv)
            plsc.store_scatter(cd1, [offs1], ov)
            plsc.store_scatter(cp1, [offs1], pos)
            co1 = co1 + jnp.take(cs1, _B15)
        return co0, co1
    zv = jnp.zeros((16,), jnp.int32)
    co0v, co1v = lax.fori_loop(0, 80, _row, (zv, zv))
    co0 = lax.reduce_max(co0v, (0,))
    co1 = lax.reduce_max(co1v, (0,))

    pltpu.sync_copy(ib, idx_hbm.at[pl.ds(w * 80, 80)])
    neg1 = jnp.full((16,), -1, jnp.int32)
    for setid, bufs, co in ((0, (ck0, cd0, cp0), co0), (1, (ck1, cd1, cp1), co1)):
        cop = (co + 127) & ~jnp.int32(127)

        def _fill(i, _, kbuf=bufs[0]):
            kbuf[pl.ds(co + i * 16, 16)] = neg1
            return 0
        lax.fori_loop(0, (cop - co + 15) >> 4, _fill, 0)
        base = pl.multiple_of(
            plsc.fetch_and_add(cnt_smem.at[setid], cop, subcore_id=0), 128)
        dst0 = c * 163840 + base
        nchd = cop >> 7

        def _dump(ch, _):
            for buf, hb in zip(bufs, (ck_hbm, cd_hbm, cp_hbm)):
                pltpu.async_copy(
                    buf.at[pl.ds(ch * 128, 128)],
                    hb.at[pl.ds(setid * 327680 + dst0 + ch * 128, 128)],
                    sem)
            return 0
        lax.fori_loop(0, nchd, _dump, 0)

        def _dumpw(ch, _):
            for buf, hb in zip(bufs, (ck_hbm, cd_hbm, cp_hbm)):
                pltpu.make_async_copy(
                    buf.at[pl.ds(ch * 128, 128)],
                    hb.at[pl.ds(setid * 327680 + dst0 + ch * 128, 128)],
                    sem).wait()
            return 0
        lax.fori_loop(0, nchd, _dumpw, 0)
    plsc.subcore_barrier()

    @pl.when(s == 0)
    def _():
        for setid in (0, 1):
            ccv[...] = jnp.full((16,), 1, jnp.int32) * cnt_smem[setid]
            pltpu.sync_copy(ccv, cc_hbm.at[setid * 2 + c])


@jax.jit
def _sc_k2(key2d, ord2d, counts):
    mesh = plsc.VectorSubcoreMesh(core_axis_name="c", subcore_axis_name="s")
    return pl.kernel(
        _k2_body,
        out_type=(
            jax.ShapeDtypeStruct((KROWS_PAD, 128), jnp.int32),
            jax.ShapeDtypeStruct((655360,), jnp.int32),
            jax.ShapeDtypeStruct((655360,), jnp.int32),
            jax.ShapeDtypeStruct((655360,), jnp.int32),
            jax.ShapeDtypeStruct((4, 16), jnp.int32),
        ),
        mesh=mesh,
        compiler_params=pltpu.CompilerParams(needs_layout_passes=False),
        scratch_types=[
            pltpu.VMEM((80, 128), jnp.int32),
            pltpu.VMEM((80, 128), jnp.int32),
            pltpu.VMEM((80, 128), jnp.int32),
            pltpu.VMEM((80, 128), jnp.int32),
            pltpu.VMEM((80, 128), jnp.int32),
            pltpu.VMEM((CCAP + 16,), jnp.int32),
            pltpu.VMEM((CCAP + 16,), jnp.int32),
            pltpu.VMEM((CCAP + 16,), jnp.int32),
            pltpu.VMEM((CCAP + 16,), jnp.int32),
            pltpu.VMEM((CCAP + 16,), jnp.int32),
            pltpu.VMEM((CCAP + 16,), jnp.int32),
            pltpu.VMEM((16,), jnp.int32),
            pltpu.SMEM((8,), jnp.int32),
            pltpu.SemaphoreType.DMA,
        ],
    )(key2d, ord2d, counts)


RNG = 40000
WCAP = 16384


def _k3_body(ck_hbm, cd_hbm, cp_hbm, cc_hbm,
             wp_hbm, wc_hbm,
             ordtab, postab, kst, dst_, pst, wbuf, ccs, wcv, sem):
    c = lax.axis_index("c")
    s = lax.axis_index("s")
    iota = lax.iota(jnp.int32, 16)
    rngv = jnp.full((16,), RNG, jnp.int32)
    _B15 = jnp.full((16,), 15, jnp.int32)
    pltpu.sync_copy(cc_hbm, ccs)

    for r in (0, 1):
        rbase = c * HALF + (s * 2 + r) * RNG

        def _init(i, _):
            ordtab[pl.ds(i * 16, 16)] = jnp.full((16,), -2**31, jnp.int32)
            postab[pl.ds(i * 16, 16)] = jnp.full((16,), 2**31 - 1, jnp.int32)
            return 0
        lax.fori_loop(0, (RNG + 16) // 16, _init, 0)

        def _stage(slot, ch, refs, bufs):
            for ref, buf in zip(refs, bufs):
                pltpu.async_copy(
                    ref.at[pl.ds(c * 327680 + slot * 163840 + ch * 2048,
                                 2048)], buf, sem)
            for ref, buf in zip(refs, bufs):
                pltpu.make_async_copy(
                    ref.at[pl.ds(c * 327680 + slot * 163840 + ch * 2048,
                                 2048)], buf, sem).wait()

        def _masks(slot, ch, v, cnt):
            kv = kst[pl.ds(v * 16, 16)]
            gi = ch * 2048 + v * 16 + iota
            lk = kv - rbase
            m = (gi < cnt) & (lk >= 0) & (lk < RNG)
            return lk, m, jnp.where(m, lk, rngv)

        def _a1_chunk(ch, cnt, slot):
            _stage(slot, ch, (ck_hbm, cd_hbm), (kst, dst_))
            nv = (jnp.minimum(cnt - ch * 2048, 2048) + 15) >> 4

            def _sweep(_):
                def _vb(v, acc):
                    lk, m, lkc = _masks(slot, ch, v, cnt)
                    ov = dst_[pl.ds(v * 16, 16)]
                    ovc = jnp.where(m, ov, jnp.int32(-2**31))
                    kk = jnp.where(m, lkc, jnp.int32(-1))
                    for d in (1, 2):
                        pidx = jnp.maximum(iota - d, 0)
                        same = kk == jnp.take(kk, pidx)
                        ovc = jnp.where(same,
                                        jnp.maximum(ovc, jnp.take(ovc, pidx)),
                                        ovc)
                    g = plsc.load_gather(ordtab, [lkc])
                    better = m & (ovc > g)
                    plsc.store_scatter(
                        ordtab, [jnp.where(better, lkc, rngv)], ovc)
                    return jnp.maximum(acc, jnp.where(better, 1, 0))
                accv = lax.fori_loop(0, nv, _vb, jnp.zeros((16,), jnp.int32))
                return lax.reduce_max(accv, (0,))
            lax.while_loop(lambda g: g > 0, _sweep, jnp.int32(1))

        def _a2_chunk(ch, cnt, slot):
            _stage(slot, ch, (ck_hbm, cd_hbm, cp_hbm), (kst, dst_, pst))
            nv = (jnp.minimum(cnt - ch * 2048, 2048) + 15) >> 4

            def _sweep(_):
                def _vb(v, acc):
                    lk, m, lkc = _masks(slot, ch, v, cnt)
                    ov = dst_[pl.ds(v * 16, 16)]
                    pv = pst[pl.ds(v * 16, 16)]
                    g0 = plsc.load_gather(ordtab, [lkc])
                    elig = m & (ov == g0)
                    pvc = jnp.where(elig, pv, jnp.int32(2**31 - 1))
                    kk = jnp.where(elig, lkc, jnp.int32(-1))
                    for d in (1, 2):
                        pidx = jnp.maximum(iota - d, 0)
                        same = kk == jnp.take(kk, pidx)
                        pvc = jnp.where(same,
                                        jnp.minimum(pvc, jnp.take(pvc, pidx)),
                                        pvc)
                    gp = plsc.load_gather(postab, [lkc])
                    better = elig & (pvc < gp)
                    plsc.store_scatter(
                        postab, [jnp.where(better, lkc, rngv)], pvc)
                    return jnp.maximum(acc, jnp.where(better, 1, 0))
                accv = lax.fori_loop(0, nv, _vb, jnp.zeros((16,), jnp.int32))
                return lax.reduce_max(accv, (0,))
            lax.while_loop(lambda g: g > 0, _sweep, jnp.int32(1))

        def _b_chunk(ch, cnt, slot, wo):
            _stage(slot, ch, (ck_hbm, cp_hbm), (kst, pst))
            nv = (jnp.minimum(cnt - ch * 2048, 2048) + 15) >> 4

            def _vb(v, wo):
                lk, m, lkc = _masks(slot, ch, v, cnt)
                pv = pst[pl.ds(v * 16, 16)]
                gp = plsc.load_gather(postab, [lkc])
                win = m & (pv == gp)
                cs = plsc.cumsum(jnp.where(win, 1, 0))
                offs = jnp.minimum(jnp.where(win, wo + cs - 1,
                                             jnp.int32(WCAP)), jnp.int32(WCAP))
                plsc.store_scatter(wbuf, [offs], pv)
                return wo + jnp.take(cs, _B15)
            return lax.fori_loop(0, nv, _vb, wo)

        for phase in ("a1", "a2"):
            for slot in (0, 1):
                cnt = lax.reduce_max(ccs[c * 2 + slot, :], (0,))
                nch = (cnt + 2047) >> 11

                def _ch(ch, _, phase=phase, slot=slot, cnt=cnt):
                    if phase == "a1":
                        _a1_chunk(ch, cnt, slot)
                    else:
                        _a2_chunk(ch, cnt, slot)
                    return 0
                lax.fori_loop(0, nch, _ch, 0)

        wo = jnp.zeros((16,), jnp.int32)
        for slot in (0, 1):
            cnt = lax.reduce_max(ccs[c * 2 + slot, :], (0,))
            nch = (cnt + 2047) >> 11

            def _ch(ch, wo, slot=slot, cnt=cnt):
                return _b_chunk(ch, cnt, slot, wo)
            wo = lax.fori_loop(0, nch, _ch, wo)

        q = (c * 16 + s) * 2 + r
        pltpu.sync_copy(wbuf.at[pl.ds(0, WCAP)], wp_hbm.at[q])
        wcv[...] = jnp.full((16,), 1, jnp.int32) * lax.reduce_max(wo, (0,))
        pltpu.sync_copy(wcv, wc_hbm.at[q])


@jax.jit
def _sc_k3(ck, cd, cp, cc):
    mesh = plsc.VectorSubcoreMesh(core_axis_name="c", subcore_axis_name="s")
    return pl.kernel(
        _k3_body,
        out_type=(
            jax.ShapeDtypeStruct((64, WCAP), jnp.int32),
            jax.ShapeDtypeStruct((64, 16), jnp.int32),
        ),
        mesh=mesh,
        compiler_params=pltpu.CompilerParams(needs_layout_passes=False),
        scratch_types=[
            pltpu.VMEM((RNG + 16,), jnp.int32),
            pltpu.VMEM((RNG + 16,), jnp.int32),
            pltpu.VMEM((2048,), jnp.int32),
            pltpu.VMEM((2048,), jnp.int32),
            pltpu.VMEM((2048,), jnp.int32),
            pltpu.VMEM((WCAP + 16,), jnp.int32),
            pltpu.VMEM((4, 16), jnp.int32),
            pltpu.VMEM((16,), jnp.int32),
            pltpu.SemaphoreType.DMA,
        ],
    )(ck, cd, cp, cc)


def _k3b_body(idx_hbm, wp_hbm, wc_hbm, out_hbm, ib, wst, wstall, wcs, sem):
    c = lax.axis_index("c")
    s = lax.axis_index("s")
    w = c * 16 + s
    iota = lax.iota(jnp.int32, 16)
    dumprow = jnp.full((16,), 80, jnp.int32)

    pltpu.sync_copy(idx_hbm.at[pl.ds(w * 80, 80)], ib.at[pl.ds(0, 80)])
    pltpu.sync_copy(wc_hbm, wcs)

    def _pf(q, _):
        pltpu.async_copy(wp_hbm.at[q, pl.ds(0, 1024)], wstall.at[q], sem)
        return 0
    lax.fori_loop(0, 64, _pf, 0)

    def _pfw(q, _):
        pltpu.make_async_copy(wp_hbm.at[q, pl.ds(0, 1024)], wstall.at[q],
                              sem).wait()
        return 0
    lax.fori_loop(0, 64, _pfw, 0)

    def _patch(q, ch, cnt, buf2d, row1d):
        nv = (jnp.minimum(cnt - ch * 1024, 1024) + 15) >> 4

        def _vb(v, _):
            if buf2d is not None:
                pv = buf2d[q, pl.ds(v * 16, 16)]
            else:
                pv = row1d[pl.ds(v * 16, 16)]
            gi = ch * 1024 + v * 16 + iota
            lp = pv - w * 10240
            m = (gi < cnt) & (lp >= 0) & (lp < 10240)
            row = jnp.where(m, lp >> 7, dumprow)
            col = lp & 127
            g = plsc.load_gather(ib, [row, col])
            plsc.store_scatter(ib, [row, col], g + 1)
            return 0
        lax.fori_loop(0, nv, _vb, 0)

    def _run(q, _):
        cnt = lax.reduce_max(wcs[q, :], (0,))
        _patch(q, jnp.int32(0), cnt, wstall, None)
        nch = (cnt + 1023) >> 10

        def _ch(ch, _):
            pltpu.sync_copy(wp_hbm.at[q, pl.ds(ch * 1024, 1024)], wst)
            _patch(q, ch, cnt, None, wst)
            return 0
        lax.fori_loop(1, nch, _ch, 0)
        return 0
    lax.fori_loop(0, 64, _run, 0)

    pltpu.sync_copy(ib.at[pl.ds(0, 80)], out_hbm.at[pl.ds(w * 80, 80)])


@jax.jit
def _sc_k3b(idx_pad, wp, wc):
    mesh = plsc.VectorSubcoreMesh(core_axis_name="c", subcore_axis_name="s")
    return pl.kernel(
        _k3b_body,
        out_type=jax.ShapeDtypeStruct((KROWS_PAD, 128), jnp.int32),
        mesh=mesh,
        compiler_params=pltpu.CompilerParams(needs_layout_passes=False),
        scratch_types=[
            pltpu.VMEM((81, 128), jnp.int32),
            pltpu.VMEM((1024,), jnp.int32),
            pltpu.VMEM((64, 1024), jnp.int32),
            pltpu.VMEM((64, 16), jnp.int32),
            pltpu.SemaphoreType.DMA,
        ],
    )(idx_pad, wp, wc)


def _gather_block(idx_ref, ehi_ref, elo_ref, out_ref):
    idxv = idx_ref[0, 0, :]
    onehot = (idxv.astype(jnp.int16)[:, None] ==
              lax.broadcasted_iota(jnp.int16, (_GB, EMB_ROWS), 1)
              ).astype(jnp.bfloat16)
    dn = (((1,), (0,)), ((), ()))
    hi = lax.dot_general(onehot, ehi_ref[...], dimension_numbers=dn,
                         preferred_element_type=jnp.float32)
    lo = lax.dot_general(onehot, elo_ref[...], dimension_numbers=dn,
                         preferred_element_type=jnp.float32)
    out_ref[0] = hi + lo


def _emb_gather(idx, emb):
    idx3 = idx.reshape(_GRID, 1, _GB)
    ehi = emb.astype(jnp.bfloat16)
    elo = (emb - ehi.astype(jnp.float32)).astype(jnp.bfloat16)
    out = pl.pallas_call(
        _gather_block,
        grid=(_GRID,),
        in_specs=[
            pl.BlockSpec((1, 1, _GB), lambda i: (i, 0, 0)),
            pl.BlockSpec((EMB_ROWS, H), lambda i: (0, 0)),
            pl.BlockSpec((EMB_ROWS, H), lambda i: (0, 0)),
        ],
        out_specs=pl.BlockSpec((1, _GB, H), lambda i: (i, 0, 0)),
        out_shape=jax.ShapeDtypeStruct((_GRID, _GB, H), jnp.float32),
    )(idx3, ehi, elo)
    return out.reshape(N, H)


def kernel(binary_states, scalars, index, emb):
    n = binary_states.shape[0]
    powers = (2.0 ** jnp.arange(S)).astype(jnp.float32)
    states_i = (2.0 * jnp.dot(binary_states, powers)).astype(jnp.int32)
    sh = states_i >> 1
    key = index.astype(jnp.int32) * jnp.int32(256) + sh

    logits = -scalars.squeeze() + 0.0
    b = lax.bitcast_convert_type(logits, jnp.int32)
    sign = jnp.int32(-2147483648)
    ordv = jnp.where(b < 0, jnp.bitwise_xor(~b, sign), b)

    key2d = jnp.concatenate(
        [key.reshape(KROWS, 128),
         jnp.full((KROWS_PAD - KROWS, 128), -1, jnp.int32)], axis=0)
    counts = _sc_counts(key2d)

    ord2d = jnp.concatenate(
        [ordv.reshape(KROWS, 128),
         jnp.zeros((KROWS_PAD - KROWS, 128), jnp.int32)], axis=0)
    idx_pad, ck, cd, cp, cc = _sc_k2(key2d, ord2d, counts)
    wp, wc = _sc_k3(ck, cd, cp, cc)
    idx_final = _sc_k3b(idx_pad, wp, wc)
    idx = idx_final.reshape(-1)[:N]
    return _emb_gather(idx, emb)


# speedup vs baseline: 1.1724x; 1.1724x over previous
import jax
import jax.numpy as jnp
from jax import lax
from jax.experimental import pallas as pl
from jax.experimental.pallas import tpu as pltpu, tpu_sc as plsc

N = 320000
S = 8
H = 128
NUM_NODES = 10000
EMB_ROWS = 2 ** (S + 1)
KEYSPACE = NUM_NODES * 256
HALF = KEYSPACE // 2
TBL = HALF + 128
KROWS = N // 128
KROWS_PAD = 2560

_GB = 4000
_GRID = N // _GB


def _counts_body(key_hbm, out_hbm, tbl, kb, idx2, ones, zb, sem):
    c = lax.axis_index("c")
    s = lax.axis_index("s")
    half = jnp.int32(HALF)
    dump = jnp.full((16,), HALF, jnp.int32)

    def _zb(i, _):
        zb[pl.ds(i * 16, 16)] = jnp.zeros((16,), jnp.int32)
        return 0
    lax.fori_loop(0, 512, _zb, 0)
    zoff = s * 80008
    zchunks = ((0, 8192), (8192, 8192), (16384, 8192), (24576, 8192),
               (32768, 8192), (40960, 8192), (49152, 8192),
               (57344, 8192), (65536, 8192), (73728, 6280))
    for off, sz in zchunks:
        pltpu.async_copy(zb.at[pl.ds(0, sz)], tbl.at[pl.ds(zoff + off, sz)],
                         sem)
    for off, sz in zchunks:
        pltpu.make_async_copy(zb.at[pl.ds(0, sz)],
                              tbl.at[pl.ds(zoff + off, sz)], sem).wait()
    for l in range(8):
        ones[pl.ds(l * 16, 16)] = jnp.ones((16,), jnp.int32)
    plsc.subcore_barrier()

    pltpu.sync_copy(key_hbm.at[pl.ds(s * 160, 160)], kb)

    def _row(j, _):
        for l in range(8):
            kv = kb[j, pl.ds(l * 16, 16)]
            lk = kv - c * half
            ok = (lk >= 0) & (lk < half)
            idx2[j, pl.ds(l * 16, 16)] = jnp.where(ok, lk, dump)
        return 0
    lax.fori_loop(0, 160, _row, 0)

    def _fire(j, _):
        pltpu.async_copy(ones, tbl.at[idx2.at[j]], sem, add=True)
        return 0
    lax.fori_loop(0, 160, _fire, 0)

    def _drain(j, _):
        pltpu.make_async_copy(ones, tbl.at[idx2.at[j]], sem).wait()
        return 0
    lax.fori_loop(0, 160, _drain, 0)
    plsc.subcore_barrier()

    pltpu.sync_copy(tbl.at[pl.ds(s * 80000, 80000)],
                    out_hbm.at[pl.ds(c * HALF + s * 80000, 80000)])


@jax.jit
def _sc_counts(key2d):
    mesh = plsc.VectorSubcoreMesh(core_axis_name="c", subcore_axis_name="s")
    return pl.kernel(
        _counts_body,
        out_type=jax.ShapeDtypeStruct((KEYSPACE,), jnp.int32),
        mesh=mesh,
        scratch_types=[
            pltpu.VMEM_SHARED((TBL,), jnp.int32),
            pltpu.VMEM((160, 128), jnp.int32),
            pltpu.VMEM((160, 128), jnp.int32),
            pltpu.VMEM((128,), jnp.int32),
            pltpu.VMEM((8192,), jnp.int32),
            pltpu.SemaphoreType.DMA,
        ],
    )(key2d)


CCAP = 10240


def _k2_body(key_hbm, ord_hbm, cnt_hbm,
             idx_hbm, ck_hbm, cd_hbm, cp_hbm, cc_hbm,
             kb, ob, cnb, cib, ib, ck0, ck1, cd0, cd1, cp0, cp1, ccv,
             cnt_smem, sem):
    c = lax.axis_index("c")
    s = lax.axis_index("s")
    w = c * 16 + s
    half = jnp.int32(HALF)
    iota = lax.iota(jnp.int32, 16)
    _B15 = jnp.full((16,), 15, jnp.int32)

    @pl.when(s == 0)
    def _():
        cnt_smem[0] = jnp.int32(0)
        cnt_smem[1] = jnp.int32(0)
    plsc.subcore_barrier()

    pltpu.sync_copy(key_hbm.at[pl.ds(w * 80, 80)], kb)
    pltpu.sync_copy(ord_hbm.at[pl.ds(w * 80, 80)], ob)

    def _cidx(j, _):
        for l in range(8):
            kv = kb[j, pl.ds(l * 16, 16)]
            cib[j, pl.ds(l * 16, 16)] = jnp.maximum(kv, 0)
        return 0
    lax.fori_loop(0, 80, _cidx, 0)

    def _fire(j, _):
        pltpu.async_copy(cnt_hbm.at[cib.at[j]], cnb.at[j], sem)
        return 0
    lax.fori_loop(0, 80, _fire, 0)

    def _drain(j, _):
        pltpu.make_async_copy(cnt_hbm.at[cib.at[j]], cnb.at[j], sem).wait()
        return 0
    lax.fori_loop(0, 80, _drain, 0)

    def _row(j, carry):
        co0, co1 = carry
        for l in range(8):
            kv = kb[j, pl.ds(l * 16, 16)]
            ov = ob[j, pl.ds(l * 16, 16)]
            cv = cnb[j, pl.ds(l * 16, 16)]
            pos = w * 10240 + j * 128 + l * 16 + iota
            valid = kv >= 0
            single = valid & (cv == 1)
            ib[j, pl.ds(l * 16, 16)] = ((kv & 255) << 1) + \
                jnp.where(single, 1, 0)
            cm = valid & (cv >= 2)
            m0 = cm & (kv < half)
            m1 = cm & (kv >= half)
            cs0 = plsc.cumsum(jnp.where(m0, 1, 0))
            offs0 = jnp.where(m0, co0 + cs0 - 1, jnp.int32(CCAP))
            plsc.store_scatter(ck0, [offs0], kv)
            plsc.store_scatter(cd0, [offs0], ov)
            plsc.store_scatter(cp0, [offs0], pos)
            co0 = co0 + jnp.take(cs0, _B15)
            cs1 = plsc.cumsum(jnp.where(m1, 1, 0))
            offs1 = jnp.where(m1, co1 + cs1 - 1, jnp.int32(CCAP))
            plsc.store_scatter(ck1, [offs1], kv)
            plsc.store_scatter(cd1, [offs1], ov)
            plsc.store_scatter(cp1, [offs1], pos)
            co1 = co1 + jnp.take(cs1, _B15)
        return co0, co1
    zv = jnp.zeros((16,), jnp.int32)
    co0v, co1v = lax.fori_loop(0, 80, _row, (zv, zv))
    co0 = lax.reduce_max(co0v, (0,))
    co1 = lax.reduce_max(co1v, (0,))

    pltpu.sync_copy(ib, idx_hbm.at[pl.ds(w * 80, 80)])
    neg1 = jnp.full((16,), -1, jnp.int32)
    for setid, bufs, co in ((0, (ck0, cd0, cp0), co0), (1, (ck1, cd1, cp1), co1)):
        cop = (co + 127) & ~jnp.int32(127)

        def _fill(i, _, kbuf=bufs[0]):
            kbuf[pl.ds(co + i * 16, 16)] = neg1
            return 0
        lax.fori_loop(0, (cop - co + 15) >> 4, _fill, 0)
        base = pl.multiple_of(
            plsc.fetch_and_add(cnt_smem.at[setid], cop, subcore_id=0), 128)
        dst0 = c * 163840 + base
        nchd = cop >> 7

        def _dump(ch, _):
            for buf, hb in zip(bufs, (ck_hbm, cd_hbm, cp_hbm)):
                pltpu.async_copy(
                    buf.at[pl.ds(ch * 128, 128)],
                    hb.at[pl.ds(setid * 327680 + dst0 + ch * 128, 128)],
                    sem)
            return 0
        lax.fori_loop(0, nchd, _dump, 0)

        def _dumpw(ch, _):
            for buf, hb in zip(bufs, (ck_hbm, cd_hbm, cp_hbm)):
                pltpu.make_async_copy(
                    buf.at[pl.ds(ch * 128, 128)],
                    hb.at[pl.ds(setid * 327680 + dst0 + ch * 128, 128)],
                    sem).wait()
            return 0
        lax.fori_loop(0, nchd, _dumpw, 0)
    plsc.subcore_barrier()

    @pl.when(s == 0)
    def _():
        for setid in (0, 1):
            ccv[...] = jnp.full((16,), 1, jnp.int32) * cnt_smem[setid]
            pltpu.sync_copy(ccv, cc_hbm.at[setid * 2 + c])


@jax.jit
def _sc_k2(key2d, ord2d, counts):
    mesh = plsc.VectorSubcoreMesh(core_axis_name="c", subcore_axis_name="s")
    return pl.kernel(
        _k2_body,
        out_type=(
            jax.ShapeDtypeStruct((KROWS_PAD, 128), jnp.int32),
            jax.ShapeDtypeStruct((655360,), jnp.int32),
            jax.ShapeDtypeStruct((655360,), jnp.int32),
            jax.ShapeDtypeStruct((655360,), jnp.int32),
            jax.ShapeDtypeStruct((4, 16), jnp.int32),
        ),
        mesh=mesh,
        compiler_params=pltpu.CompilerParams(needs_layout_passes=False),
        scratch_types=[
            pltpu.VMEM((80, 128), jnp.int32),
            pltpu.VMEM((80, 128), jnp.int32),
            pltpu.VMEM((80, 128), jnp.int32),
            pltpu.VMEM((80, 128), jnp.int32),
            pltpu.VMEM((80, 128), jnp.int32),
            pltpu.VMEM((CCAP + 16,), jnp.int32),
            pltpu.VMEM((CCAP + 16,), jnp.int32),
            pltpu.VMEM((CCAP + 16,), jnp.int32),
            pltpu.VMEM((CCAP + 16,), jnp.int32),
            pltpu.VMEM((CCAP + 16,), jnp.int32),
            pltpu.VMEM((CCAP + 16,), jnp.int32),
            pltpu.VMEM((16,), jnp.int32),
            pltpu.SMEM((8,), jnp.int32),
            pltpu.SemaphoreType.DMA,
        ],
    )(key2d, ord2d, counts)


RNG = 40000
WCAP = 16384


def _k3_body(ck_hbm, cd_hbm, cp_hbm, cc_hbm,
             wp_hbm, wc_hbm,
             ordtab, postab, kst, dst_, pst, wbuf, ccs, wcv, sem):
    c = lax.axis_index("c")
    s = lax.axis_index("s")
    iota = lax.iota(jnp.int32, 16)
    rngv = jnp.full((16,), RNG, jnp.int32)
    _B15 = jnp.full((16,), 15, jnp.int32)
    pltpu.sync_copy(cc_hbm, ccs)

    for r in (0, 1):
        rbase = c * HALF + (s * 2 + r) * RNG

        def _init(i, _):
            ordtab[pl.ds(i * 16, 16)] = jnp.full((16,), -2**31, jnp.int32)
            postab[pl.ds(i * 16, 16)] = jnp.full((16,), 2**31 - 1, jnp.int32)
            return 0
        lax.fori_loop(0, (RNG + 16) // 16, _init, 0)

        def _stage(slot, ch, refs, bufs):
            for ref, buf in zip(refs, bufs):
                pltpu.async_copy(
                    ref.at[pl.ds(c * 327680 + slot * 163840 + ch * 2048,
                                 2048)], buf, sem)
            for ref, buf in zip(refs, bufs):
                pltpu.make_async_copy(
                    ref.at[pl.ds(c * 327680 + slot * 163840 + ch * 2048,
                                 2048)], buf, sem).wait()

        def _masks(slot, ch, v, cnt):
            kv = kst[pl.ds(v * 16, 16)]
            gi = ch * 2048 + v * 16 + iota
            lk = kv - rbase
            m = (gi < cnt) & (lk >= 0) & (lk < RNG)
            return lk, m, jnp.where(m, lk, rngv)

        def _a1_chunk(ch, cnt, slot):
            _stage(slot, ch, (ck_hbm, cd_hbm), (kst, dst_))
            nv = (jnp.minimum(cnt - ch * 2048, 2048) + 15) >> 4

            def _sweep(_):
                def _vb(v, acc):
                    lk, m, lkc = _masks(slot, ch, v, cnt)
                    ov = dst_[pl.ds(v * 16, 16)]
                    g = plsc.load_gather(ordtab, [lkc])
                    better = m & (ov > g)
                    plsc.store_scatter(
                        ordtab, [jnp.where(better, lkc, rngv)], ov)
                    g2 = plsc.load_gather(ordtab, [lkc])
                    conf = m & (ov > g2)
                    return jnp.maximum(acc, jnp.where(conf, 1, 0))
                accv = lax.fori_loop(0, nv, _vb, jnp.zeros((16,), jnp.int32))
                return lax.reduce_max(accv, (0,))
            lax.while_loop(lambda g: g > 0, _sweep, jnp.int32(1))

        def _a2_chunk(ch, cnt, slot):
            _stage(slot, ch, (ck_hbm, cd_hbm, cp_hbm), (kst, dst_, pst))
            nv = (jnp.minimum(cnt - ch * 2048, 2048) + 15) >> 4

            def _sweep(_):
                def _vb(v, acc):
                    lk, m, lkc = _masks(slot, ch, v, cnt)
                    ov = dst_[pl.ds(v * 16, 16)]
                    pv = pst[pl.ds(v * 16, 16)]
                    g0 = plsc.load_gather(ordtab, [lkc])
                    gp = plsc.load_gather(postab, [lkc])
                    better = m & (ov == g0) & (pv < gp)
                    plsc.store_scatter(
                        postab, [jnp.where(better, lkc, rngv)], pv)
                    gp2 = plsc.load_gather(postab, [lkc])
                    conf = m & (ov == g0) & (pv < gp2)
                    return jnp.maximum(acc, jnp.where(conf, 1, 0))
                accv = lax.fori_loop(0, nv, _vb, jnp.zeros((16,), jnp.int32))
                return lax.reduce_max(accv, (0,))
            lax.while_loop(lambda g: g > 0, _sweep, jnp.int32(1))

        def _b_chunk(ch, cnt, slot, wo):
            _stage(slot, ch, (ck_hbm, cp_hbm), (kst, pst))
            nv = (jnp.minimum(cnt - ch * 2048, 2048) + 15) >> 4

            def _vb(v, wo):
                lk, m, lkc = _masks(slot, ch, v, cnt)
                pv = pst[pl.ds(v * 16, 16)]
                gp = plsc.load_gather(postab, [lkc])
                win = m & (pv == gp)
                cs = plsc.cumsum(jnp.where(win, 1, 0))
                offs = jnp.minimum(jnp.where(win, wo + cs - 1,
                                             jnp.int32(WCAP)), jnp.int32(WCAP))
                plsc.store_scatter(wbuf, [offs], pv)
                return wo + jnp.take(cs, _B15)
            return lax.fori_loop(0, nv, _vb, wo)

        for phase in ("a1", "a2"):
            for slot in (0, 1):
                cnt = lax.reduce_max(ccs[c * 2 + slot, :], (0,))
                nch = (cnt + 2047) >> 11

                def _ch(ch, _, phase=phase, slot=slot, cnt=cnt):
                    if phase == "a1":
                        _a1_chunk(ch, cnt, slot)
                    else:
                        _a2_chunk(ch, cnt, slot)
                    return 0
                lax.fori_loop(0, nch, _ch, 0)

        wo = jnp.zeros((16,), jnp.int32)
        for slot in (0, 1):
            cnt = lax.reduce_max(ccs[c * 2 + slot, :], (0,))
            nch = (cnt + 2047) >> 11

            def _ch(ch, wo, slot=slot, cnt=cnt):
                return _b_chunk(ch, cnt, slot, wo)
            wo = lax.fori_loop(0, nch, _ch, wo)

        q = (c * 16 + s) * 2 + r
        pltpu.sync_copy(wbuf.at[pl.ds(0, WCAP)], wp_hbm.at[q])
        wcv[...] = jnp.full((16,), 1, jnp.int32) * lax.reduce_max(wo, (0,))
        pltpu.sync_copy(wcv, wc_hbm.at[q])


@jax.jit
def _sc_k3(ck, cd, cp, cc):
    mesh = plsc.VectorSubcoreMesh(core_axis_name="c", subcore_axis_name="s")
    return pl.kernel(
        _k3_body,
        out_type=(
            jax.ShapeDtypeStruct((64, WCAP), jnp.int32),
            jax.ShapeDtypeStruct((64, 16), jnp.int32),
        ),
        mesh=mesh,
        compiler_params=pltpu.CompilerParams(needs_layout_passes=False),
        scratch_types=[
            pltpu.VMEM((RNG + 16,), jnp.int32),
            pltpu.VMEM((RNG + 16,), jnp.int32),
            pltpu.VMEM((2048,), jnp.int32),
            pltpu.VMEM((2048,), jnp.int32),
            pltpu.VMEM((2048,), jnp.int32),
            pltpu.VMEM((WCAP + 16,), jnp.int32),
            pltpu.VMEM((4, 16), jnp.int32),
            pltpu.VMEM((16,), jnp.int32),
            pltpu.SemaphoreType.DMA,
        ],
    )(ck, cd, cp, cc)


def _k3b_body(idx_hbm, wp_hbm, wc_hbm, out_hbm, ib, wst, wstall, wcs, sem):
    c = lax.axis_index("c")
    s = lax.axis_index("s")
    w = c * 16 + s
    iota = lax.iota(jnp.int32, 16)
    dumprow = jnp.full((16,), 80, jnp.int32)

    pltpu.sync_copy(idx_hbm.at[pl.ds(w * 80, 80)], ib.at[pl.ds(0, 80)])
    pltpu.sync_copy(wc_hbm, wcs)

    def _pf(q, _):
        pltpu.async_copy(wp_hbm.at[q, pl.ds(0, 1024)], wstall.at[q], sem)
        return 0
    lax.fori_loop(0, 64, _pf, 0)

    def _pfw(q, _):
        pltpu.make_async_copy(wp_hbm.at[q, pl.ds(0, 1024)], wstall.at[q],
                              sem).wait()
        return 0
    lax.fori_loop(0, 64, _pfw, 0)

    def _patch(q, ch, cnt, buf2d, row1d):
        nv = (jnp.minimum(cnt - ch * 1024, 1024) + 15) >> 4

        def _vb(v, _):
            if buf2d is not None:
                pv = buf2d[q, pl.ds(v * 16, 16)]
            else:
                pv = row1d[pl.ds(v * 16, 16)]
            gi = ch * 1024 + v * 16 + iota
            lp = pv - w * 10240
            m = (gi < cnt) & (lp >= 0) & (lp < 10240)
            row = jnp.where(m, lp >> 7, dumprow)
            col = lp & 127
            g = plsc.load_gather(ib, [row, col])
            plsc.store_scatter(ib, [row, col], g + 1)
            return 0
        lax.fori_loop(0, nv, _vb, 0)

    def _run(q, _):
        cnt = lax.reduce_max(wcs[q, :], (0,))
        _patch(q, jnp.int32(0), cnt, wstall, None)
        nch = (cnt + 1023) >> 10

        def _ch(ch, _):
            pltpu.sync_copy(wp_hbm.at[q, pl.ds(ch * 1024, 1024)], wst)
            _patch(q, ch, cnt, None, wst)
            return 0
        lax.fori_loop(1, nch, _ch, 0)
        return 0
    lax.fori_loop(0, 64, _run, 0)

    pltpu.sync_copy(ib.at[pl.ds(0, 80)], out_hbm.at[pl.ds(w * 80, 80)])


@jax.jit
def _sc_k3b(idx_pad, wp, wc):
    mesh = plsc.VectorSubcoreMesh(core_axis_name="c", subcore_axis_name="s")
    return pl.kernel(
        _k3b_body,
        out_type=jax.ShapeDtypeStruct((KROWS_PAD, 128), jnp.int32),
        mesh=mesh,
        compiler_params=pltpu.CompilerParams(needs_layout_passes=False),
        scratch_types=[
            pltpu.VMEM((81, 128), jnp.int32),
            pltpu.VMEM((1024,), jnp.int32),
            pltpu.VMEM((64, 1024), jnp.int32),
            pltpu.VMEM((64, 16), jnp.int32),
            pltpu.SemaphoreType.DMA,
        ],
    )(idx_pad, wp, wc)


def _gather_block(idx_ref, ehi_ref, elo_ref, out_ref):
    idxv = idx_ref[0, 0, :]
    onehot = (idxv.astype(jnp.int16)[:, None] ==
              lax.broadcasted_iota(jnp.int16, (_GB, EMB_ROWS), 1)
              ).astype(jnp.bfloat16)
    dn = (((1,), (0,)), ((), ()))
    hi = lax.dot_general(onehot, ehi_ref[...], dimension_numbers=dn,
                         preferred_element_type=jnp.float32)
    lo = lax.dot_general(onehot, elo_ref[...], dimension_numbers=dn,
                         preferred_element_type=jnp.float32)
    out_ref[0] = hi + lo


def _emb_gather(idx, emb):
    idx3 = idx.reshape(_GRID, 1, _GB)
    ehi = emb.astype(jnp.bfloat16)
    elo = (emb - ehi.astype(jnp.float32)).astype(jnp.bfloat16)
    out = pl.pallas_call(
        _gather_block,
        grid=(_GRID,),
        in_specs=[
            pl.BlockSpec((1, 1, _GB), lambda i: (i, 0, 0)),
            pl.BlockSpec((EMB_ROWS, H), lambda i: (0, 0)),
            pl.BlockSpec((EMB_ROWS, H), lambda i: (0, 0)),
        ],
        out_specs=pl.BlockSpec((1, _GB, H), lambda i: (i, 0, 0)),
        out_shape=jax.ShapeDtypeStruct((_GRID, _GB, H), jnp.float32),
    )(idx3, ehi, elo)
    return out.reshape(N, H)


def kernel(binary_states, scalars, index, emb):
    n = binary_states.shape[0]
    powers = (2.0 ** jnp.arange(S)).astype(jnp.float32)
    states_i = (2.0 * jnp.dot(binary_states, powers)).astype(jnp.int32)
    sh = states_i >> 1
    key = index.astype(jnp.int32) * jnp.int32(256) + sh

    logits = -scalars.squeeze() + 0.0
    b = lax.bitcast_convert_type(logits, jnp.int32)
    sign = jnp.int32(-2147483648)
    ordv = jnp.where(b < 0, jnp.bitwise_xor(~b, sign), b)

    key2d = jnp.concatenate(
        [key.reshape(KROWS, 128),
         jnp.full((KROWS_PAD - KROWS, 128), -1, jnp.int32)], axis=0)
    counts = _sc_counts(key2d)

    ord2d = jnp.concatenate(
        [ordv.reshape(KROWS, 128),
         jnp.zeros((KROWS_PAD - KROWS, 128), jnp.int32)], axis=0)
    idx_pad, ck, cd, cp, cc = _sc_k2(key2d, ord2d, counts)
    wp, wc = _sc_k3(ck, cd, cp, cc)
    idx_final = _sc_k3b(idx_pad, wp, wc)
    idx = idx_final.reshape(-1)[:N]
    return _emb_gather(idx, emb)
